# baseline (speedup 1.0000x reference)
import functools

import jax
import jax.numpy as jnp
from jax import lax
from jax.experimental import pallas as pl
from jax.experimental.pallas import tpu as pltpu
from jax.experimental.pallas import tpu_sc as plsc

TOPK_ = 32
G_ = 128
BM_ = 256
BN_ = 4096
LN_ = 16


def _enc_body(x_ref, b_ref, w_ref, p_ref, bmax_ref, *, bm, bn, g):
    x = x_ref[...] - b_ref[...]
    p = jax.lax.dot_general(
        x, w_ref[...],
        dimension_numbers=(((1,), (1,)), ((), ())),
        preferred_element_type=jnp.float32,
    )
    p_ref[...] = p
    nbt = bn // g
    cols = [jnp.max(p[:, t * g:(t + 1) * g], axis=1, keepdims=True)
            for t in range(nbt)]
    bmax_ref[...] = jnp.concatenate(cols, axis=1).reshape(1, bm, nbt)


def _encode(embed, bias, W_enc):
    B, E = embed.shape
    F = W_enc.shape[0]
    bm, bn, g = BM_, BN_, G_
    nb, nf = B // bm, F // bn
    nbt = bn // g
    body = functools.partial(_enc_body, bm=bm, bn=bn, g=g)
    project, bmax = pl.pallas_call(
        body,
        grid=(nf, nb),
        in_specs=[
            pl.BlockSpec((bm, E), lambda j, i: (i, 0)),
            pl.BlockSpec((1, E), lambda j, i: (0, 0)),
            pl.BlockSpec((bn, E), lambda j, i: (j, 0)),
        ],
        out_specs=[
            pl.BlockSpec((bm, bn), lambda j, i: (i, j)),
            pl.BlockSpec((1, bm, nbt), lambda j, i: (j, i, 0)),
        ],
        out_shape=[
            jax.ShapeDtypeStruct((B, F), jnp.float32),
            jax.ShapeDtypeStruct((nf, B, nbt), jnp.float32),
        ],
    )(embed, bias.reshape(1, -1), W_enc)
    return project, bmax.transpose(1, 0, 2).reshape(B, F // g)


def _pick_body(bm_ref, bids_ref, *, bm, nbuk):
    v = bm_ref[...]
    idx = lax.broadcasted_iota(jnp.int32, (bm, nbuk), 1)
    outs = []
    for _it in range(TOPK_):
        m = jnp.max(v, axis=1, keepdims=True)
        pos = jnp.min(jnp.where(v >= m, idx, nbuk), axis=1, keepdims=True)
        outs.append(pos)
        v = jnp.where(idx == pos, -jnp.inf, v)
    bids_ref[...] = jnp.concatenate(outs, axis=1)


def _pick_buckets(bmax):
    B, nbuk = bmax.shape
    bm = BM_
    body = functools.partial(_pick_body, bm=bm, nbuk=nbuk)
    return pl.pallas_call(
        body,
        grid=(B // bm,),
        in_specs=[pl.BlockSpec((bm, nbuk), lambda i: (i, 0))],
        out_specs=pl.BlockSpec((bm, TOPK_), lambda i: (i, 0)),
        out_shape=jax.ShapeDtypeStruct((B, TOPK_), jnp.int32),
    )(bmax)


def _gather_cands(project_rows, fidx3):
    NW, NCH, CH = fidx3.shape
    G = project_rows.shape[1]
    N = NW * NCH * CH
    mesh = plsc.VectorSubcoreMesh(core_axis_name="c", subcore_axis_name="s")

    @functools.partial(
        pl.kernel, mesh=mesh,
        out_type=jax.ShapeDtypeStruct((N, G), jnp.float32),
        scratch_types=[
            pltpu.VMEM((NCH, CH), jnp.int32),
            pltpu.VMEM((CH, G), jnp.float32),
            pltpu.VMEM((CH, G), jnp.float32),
            pltpu.SemaphoreType.DMA,
            pltpu.SemaphoreType.DMA,
        ],
    )
    def k(table_hbm, idx_hbm, out_hbm, idx_v, buf0, buf1, sem0, sem1):
        wid = lax.axis_index("s") * 2 + lax.axis_index("c")
        base = wid * (NCH * CH)
        pltpu.sync_copy(idx_hbm.at[wid], idx_v)
        bufs = (buf0, buf1)
        sems = (sem0, sem1)
        prev = None
        for ch in range(NCH):
            cp = pltpu.async_copy(
                table_hbm.at[idx_v.at[ch]], bufs[ch % 2], sems[ch % 2])
            if prev is not None:
                pch, pcp = prev
                pcp.wait()
                pltpu.sync_copy(bufs[pch % 2],
                                out_hbm.at[pl.ds(base + pch * CH, CH)])
            prev = (ch, cp)
        pch, pcp = prev
        pcp.wait()
        pltpu.sync_copy(bufs[pch % 2], out_hbm.at[pl.ds(base + pch * CH, CH)])

    return k(project_rows, fidx3)


def _sel_body(c_ref, bids_ref, w_ref, f_ref, *, bm, nc, g):
    c = c_ref[...]
    bid = bids_ref[...]
    iota = lax.broadcasted_iota(jnp.int32, (bm, nc), 1)
    iota32 = lax.broadcasted_iota(jnp.int32, (bm, TOPK_), 1)
    shift = {32: 5, 64: 6, 128: 7}[g]
    wcols, fcols = [], []
    for _it in range(TOPK_):
        m = jnp.max(c, axis=1, keepdims=True)
        pos = jnp.min(jnp.where(c >= m, iota, nc), axis=1, keepdims=True)
        slot_star = lax.shift_right_logical(pos, shift)
        lane_star = jnp.bitwise_and(pos, g - 1)
        bsel = jnp.max(jnp.where(iota32 == slot_star, bid, -1), axis=1,
                       keepdims=True)
        wcols.append(jnp.broadcast_to(m, (bm, LN_)))
        fcols.append(bsel * g + lane_star)
        c = jnp.where(iota == pos, -jnp.inf, c)
    w_ref[...] = jnp.concatenate(wcols, axis=1)
    f_ref[...] = jnp.concatenate(fcols, axis=1)


def _select(cands, bids):
    B, nc = cands.shape
    bm = BM_
    body = functools.partial(_sel_body, bm=bm, nc=nc, g=G_)
    return pl.pallas_call(
        body,
        grid=(B // bm,),
        in_specs=[
            pl.BlockSpec((bm, nc), lambda i: (i, 0)),
            pl.BlockSpec((bm, TOPK_), lambda i: (i, 0)),
        ],
        out_specs=[
            pl.BlockSpec((bm, TOPK_ * LN_), lambda i: (i, 0)),
            pl.BlockSpec((bm, TOPK_), lambda i: (i, 0)),
        ],
        out_shape=[
            jax.ShapeDtypeStruct((B, TOPK_ * LN_), jnp.float32),
            jax.ShapeDtypeStruct((B, TOPK_), jnp.int32),
        ],
    )(cands, bids)


def _decode(W_dec, feats, wexp, bias):
    F, E = W_dec.shape
    B, K = feats.shape
    NW = 32
    bper = B // NW
    EC = E // LN_
    hper = F // 16
    KL = K * LN_
    mesh = plsc.VectorSubcoreMesh(core_axis_name="c", subcore_axis_name="s")

    @functools.partial(
        pl.kernel, mesh=mesh,
        out_type=[
            jax.ShapeDtypeStruct((B, E), jnp.float32),
            jax.ShapeDtypeStruct((2, F), jnp.int32),
        ],
        scratch_types=[
            pltpu.VMEM((bper, K), jnp.int32),
            pltpu.VMEM((K, bper), jnp.int32),
            pltpu.VMEM((K, E), jnp.float32),
            pltpu.VMEM((K, E), jnp.float32),
            pltpu.VMEM((KL,), jnp.float32),
            pltpu.VMEM((KL,), jnp.float32),
            pltpu.VMEM((E,), jnp.float32),
            pltpu.VMEM((2, E), jnp.float32),
            pltpu.VMEM((bper,), jnp.int32),
            pltpu.VMEM((hper,), jnp.int32),
            pltpu.VMEM_SHARED((F,), jnp.int32),
            pltpu.SemaphoreType.DMA,
            pltpu.SemaphoreType.DMA,
            pltpu.SemaphoreType.DMA,
            pltpu.SemaphoreType.DMA,
        ],
    )
    def k(wdec_hbm, feats_hbm, featsh_hbm, wexp_hbm, bias_hbm,
          recon_hbm, counts_hbm,
          fe_v, fh_v, buf0, buf1, wrow0, wrow1, bias_v, outbuf, ones, zbuf,
          hist, sem0, sem1, wsem0, wsem1):
        cid = lax.axis_index("c")
        sid = lax.axis_index("s")
        wid = sid * 2 + cid

        def zb(i, c):
            zbuf[pl.ds(i * 16, 16)] = jnp.zeros((16,), jnp.int32)
            return c
        lax.fori_loop(0, hper // 16, zb, 0)
        pltpu.sync_copy(zbuf, hist.at[pl.ds(sid * hper, hper)])
        for i in range(bper // 16):
            ones[pl.ds(i * 16, 16)] = jnp.ones((16,), jnp.int32)
        pltpu.sync_copy(featsh_hbm.at[wid], fh_v)
        plsc.subcore_barrier()

        def hb(c, carry):
            pltpu.sync_copy(ones, hist.at[fh_v.at[c]], add=True)
            return carry
        lax.fori_loop(0, K, hb, 0)
        plsc.subcore_barrier()
        pltpu.sync_copy(hist.at[pl.ds(sid * hper, hper)],
                        counts_hbm.at[cid, pl.ds(sid * hper, hper)])

        pltpu.sync_copy(feats_hbm.at[wid], fe_v)
        pltpu.sync_copy(bias_hbm, bias_v)
        bufs = (buf0, buf1)
        sems = (sem0, sem1)
        wrows = (wrow0, wrow1)
        wsems = (wsem0, wsem1)

        def issue(b, p):
            pltpu.async_copy(wdec_hbm.at[fe_v.at[b]], bufs[p], sems[p])
            pltpu.async_copy(wexp_hbm.at[wid, b], wrows[p], wsems[p])

        def wait(p):
            pltpu.make_async_copy(wdec_hbm.at[pl.ds(0, K)], bufs[p],
                                  sems[p]).wait()
            pltpu.make_async_copy(wexp_hbm.at[0, 0], wrows[p],
                                  wsems[p]).wait()

        def compute(b, p):
            buf = bufs[p]
            wrow = wrows[p]

            def ebody(e, carry):
                sl = pl.ds(e * 16, 16)
                acc = bias_v[sl]
                for kk in range(K):
                    acc = acc + wrow[pl.ds(kk * LN_, LN_)] * buf[kk, sl]
                outbuf[p, sl] = acc
                return carry
            lax.fori_loop(0, EC, ebody, 0)
            pltpu.sync_copy(outbuf.at[p], recon_hbm.at[wid * bper + b])

        issue(0, 0)

        def body(t, carry):
            b0 = 2 * t
            issue(b0 + 1, 1)
            wait(0)
            compute(b0, 0)

            @pl.when(b0 + 2 < bper)
            def _():
                issue(b0 + 2, 0)
            wait(1)
            compute(b0 + 1, 1)
            return carry
        lax.fori_loop(0, bper // 2, body, 0)

    return k(W_dec, feats.reshape(NW, bper, K), feats.reshape(NW, K, bper),
             wexp.reshape(NW, bper, KL), bias)


def _sum_counts(counts2):
    F = counts2.shape[1]

    def body(c_ref, o_ref):
        o_ref[...] = c_ref[0:1, :] + c_ref[1:2, :]

    return pl.pallas_call(
        body,
        out_shape=jax.ShapeDtypeStruct((1, F), jnp.int32),
    )(counts2).reshape(F)


def kernel(embed, bias, W_enc, W_dec):
    B, E = embed.shape
    F = W_enc.shape[0]
    project, bmax = _encode(embed, bias, W_enc)
    bids = _pick_buckets(bmax)
    nbuk = F // G_
    fidx = (bids + jnp.arange(B, dtype=jnp.int32)[:, None] * nbuk).reshape(-1)
    NW, CH = 32, 128
    NCH = fidx.shape[0] // (NW * CH)
    cands_flat = _gather_cands(project.reshape(B * nbuk, G_),
                               fidx.reshape(NW, NCH, CH))
    cands = cands_flat.reshape(B, TOPK_ * G_)
    wexp, feats = _select(cands, bids)
    recon, counts2 = _decode(W_dec, feats, wexp, bias)
    total = _sum_counts(counts2)
    return recon, total


# speedup vs baseline: 1.0359x; 1.0359x over previous
import functools

import jax
import jax.numpy as jnp
from jax import lax
from jax.experimental import pallas as pl
from jax.experimental.pallas import tpu as pltpu
from jax.experimental.pallas import tpu_sc as plsc

TOPK_ = 32
G_ = 128
BM_ = 256
BN_ = 4096
LN_ = 16


def _enc_body(x_ref, b_ref, w_ref, p_ref, bidsT_ref, bmax_scr, *, nf, bm, bn, g):
    j = pl.program_id(0)
    i = pl.program_id(1)
    x = x_ref[...] - b_ref[...]
    p = jax.lax.dot_general(
        x, w_ref[...],
        dimension_numbers=(((1,), (1,)), ((), ())),
        preferred_element_type=jnp.float32,
    )
    p_ref[...] = p

    nbt = bn // g
    rows = [jnp.max(p[:, t * g:(t + 1) * g], axis=1).reshape(1, bm)
            for t in range(nbt)]
    bmax_scr[pl.ds(j * nbt, nbt), pl.ds(i * bm, bm)] = jnp.concatenate(rows, axis=0)

    @pl.when(j == nf - 1)
    def _():
        nbuk = nf * nbt
        v = bmax_scr[:, pl.ds(i * bm, bm)]
        idx = jax.lax.broadcasted_iota(jnp.int32, (nbuk, bm), 0)
        outs = []
        for _it in range(TOPK_):
            m = jnp.max(v, axis=0, keepdims=True)
            pos = jnp.min(jnp.where(v >= m, idx, nbuk), axis=0, keepdims=True)
            outs.append(pos)
            v = jnp.where(idx == pos, -jnp.inf, v)
        bidsT_ref[...] = jnp.concatenate(outs, axis=0).astype(jnp.int32)


def _encode(embed, bias, W_enc):
    B, E = embed.shape
    F = W_enc.shape[0]
    bm, bn, g = BM_, BN_, G_
    nb, nf = B // bm, F // bn
    body = functools.partial(_enc_body, nf=nf, bm=bm, bn=bn, g=g)
    project, bidsT = pl.pallas_call(
        body,
        grid=(nf, nb),
        in_specs=[
            pl.BlockSpec((bm, E), lambda j, i: (i, 0)),
            pl.BlockSpec((1, E), lambda j, i: (0, 0)),
            pl.BlockSpec((bn, E), lambda j, i: (j, 0)),
        ],
        out_specs=[
            pl.BlockSpec((bm, bn), lambda j, i: (i, j)),
            pl.BlockSpec((TOPK_, bm), lambda j, i: (0, i)),
        ],
        out_shape=[
            jax.ShapeDtypeStruct((B, F), jnp.float32),
            jax.ShapeDtypeStruct((TOPK_, B), jnp.int32),
        ],
        scratch_shapes=[pltpu.VMEM((F // g, B), jnp.float32)],
    )(embed, bias.reshape(1, -1), W_enc)
    return project, bidsT.T


def _gather_cands(project_rows, fidx3):
    NW, NCH, CH = fidx3.shape
    G = project_rows.shape[1]
    N = NW * NCH * CH
    mesh = plsc.VectorSubcoreMesh(core_axis_name="c", subcore_axis_name="s")

    @functools.partial(
        pl.kernel, mesh=mesh,
        out_type=jax.ShapeDtypeStruct((N, G), jnp.float32),
        scratch_types=[
            pltpu.VMEM((NCH, CH), jnp.int32),
            pltpu.VMEM((CH, G), jnp.float32),
            pltpu.VMEM((CH, G), jnp.float32),
            pltpu.SemaphoreType.DMA,
            pltpu.SemaphoreType.DMA,
        ],
    )
    def k(table_hbm, idx_hbm, out_hbm, idx_v, buf0, buf1, sem0, sem1):
        wid = lax.axis_index("s") * 2 + lax.axis_index("c")
        base = wid * (NCH * CH)
        pltpu.sync_copy(idx_hbm.at[wid], idx_v)
        bufs = (buf0, buf1)
        sems = (sem0, sem1)
        prev = None
        for ch in range(NCH):
            cp = pltpu.async_copy(
                table_hbm.at[idx_v.at[ch]], bufs[ch % 2], sems[ch % 2])
            if prev is not None:
                pch, pcp = prev
                pcp.wait()
                pltpu.sync_copy(bufs[pch % 2],
                                out_hbm.at[pl.ds(base + pch * CH, CH)])
            prev = (ch, cp)
        pch, pcp = prev
        pcp.wait()
        pltpu.sync_copy(bufs[pch % 2], out_hbm.at[pl.ds(base + pch * CH, CH)])

    return k(project_rows, fidx3)


def _sel_body(c_ref, bids_ref, w_ref, f_ref, *, bm, nc, g):
    c = c_ref[...]
    bid = bids_ref[...]
    iota = lax.broadcasted_iota(jnp.int32, (bm, nc), 1)
    iota32 = lax.broadcasted_iota(jnp.int32, (bm, TOPK_), 1)
    shift = {32: 5, 64: 6, 128: 7}[g]
    wcols, fcols = [], []
    for _it in range(TOPK_):
        m = jnp.max(c, axis=1, keepdims=True)
        pos = jnp.min(jnp.where(c >= m, iota, nc), axis=1, keepdims=True)
        slot_star = lax.shift_right_logical(pos, shift)
        lane_star = jnp.bitwise_and(pos, g - 1)
        bsel = jnp.max(jnp.where(iota32 == slot_star, bid, -1), axis=1,
                       keepdims=True)
        wcols.append(jnp.broadcast_to(m, (bm, LN_)))
        fcols.append(bsel * g + lane_star)
        c = jnp.where(iota == pos, -jnp.inf, c)
    w_ref[...] = jnp.concatenate(wcols, axis=1)
    f_ref[...] = jnp.concatenate(fcols, axis=1)


def _select(cands, bids):
    B, nc = cands.shape
    bm = BM_
    body = functools.partial(_sel_body, bm=bm, nc=nc, g=G_)
    return pl.pallas_call(
        body,
        grid=(B // bm,),
        in_specs=[
            pl.BlockSpec((bm, nc), lambda i: (i, 0)),
            pl.BlockSpec((bm, TOPK_), lambda i: (i, 0)),
        ],
        out_specs=[
            pl.BlockSpec((bm, TOPK_ * LN_), lambda i: (i, 0)),
            pl.BlockSpec((bm, TOPK_), lambda i: (i, 0)),
        ],
        out_shape=[
            jax.ShapeDtypeStruct((B, TOPK_ * LN_), jnp.float32),
            jax.ShapeDtypeStruct((B, TOPK_), jnp.int32),
        ],
    )(cands, bids)


def _decode(W_dec, feats, wexp, bias):
    F, E = W_dec.shape
    B, K = feats.shape
    NW = 32
    bper = B // NW
    EC = E // LN_
    hper = F // 16
    KL = K * LN_
    mesh = plsc.VectorSubcoreMesh(core_axis_name="c", subcore_axis_name="s")

    @functools.partial(
        pl.kernel, mesh=mesh,
        out_type=[
            jax.ShapeDtypeStruct((B, E), jnp.float32),
            jax.ShapeDtypeStruct((2, F), jnp.int32),
        ],
        scratch_types=[
            pltpu.VMEM((bper, K), jnp.int32),
            pltpu.VMEM((K, bper), jnp.int32),
            pltpu.VMEM((K, E), jnp.float32),
            pltpu.VMEM((K, E), jnp.float32),
            pltpu.VMEM((KL,), jnp.float32),
            pltpu.VMEM((KL,), jnp.float32),
            pltpu.VMEM((E,), jnp.float32),
            pltpu.VMEM((2, E), jnp.float32),
            pltpu.VMEM((bper,), jnp.int32),
            pltpu.VMEM((hper,), jnp.int32),
            pltpu.VMEM_SHARED((F,), jnp.int32),
            pltpu.SemaphoreType.DMA,
            pltpu.SemaphoreType.DMA,
            pltpu.SemaphoreType.DMA,
            pltpu.SemaphoreType.DMA,
        ],
    )
    def k(wdec_hbm, feats_hbm, featsh_hbm, wexp_hbm, bias_hbm,
          recon_hbm, counts_hbm,
          fe_v, fh_v, buf0, buf1, wrow0, wrow1, bias_v, outbuf, ones, zbuf,
          hist, sem0, sem1, wsem0, wsem1):
        cid = lax.axis_index("c")
        sid = lax.axis_index("s")
        wid = sid * 2 + cid

        def zb(i, c):
            zbuf[pl.ds(i * 16, 16)] = jnp.zeros((16,), jnp.int32)
            return c
        lax.fori_loop(0, hper // 16, zb, 0)
        pltpu.sync_copy(zbuf, hist.at[pl.ds(sid * hper, hper)])
        for i in range(bper // 16):
            ones[pl.ds(i * 16, 16)] = jnp.ones((16,), jnp.int32)
        pltpu.sync_copy(featsh_hbm.at[wid], fh_v)
        plsc.subcore_barrier()

        def hb(c, carry):
            pltpu.sync_copy(ones, hist.at[fh_v.at[c]], add=True)
            return carry
        lax.fori_loop(0, K, hb, 0)
        plsc.subcore_barrier()
        pltpu.sync_copy(hist.at[pl.ds(sid * hper, hper)],
                        counts_hbm.at[cid, pl.ds(sid * hper, hper)])

        pltpu.sync_copy(feats_hbm.at[wid], fe_v)
        pltpu.sync_copy(bias_hbm, bias_v)
        bufs = (buf0, buf1)
        sems = (sem0, sem1)
        wrows = (wrow0, wrow1)
        wsems = (wsem0, wsem1)

        def issue(b, p):
            pltpu.async_copy(wdec_hbm.at[fe_v.at[b]], bufs[p], sems[p])
            pltpu.async_copy(wexp_hbm.at[wid, b], wrows[p], wsems[p])

        def wait(p):
            pltpu.make_async_copy(wdec_hbm.at[pl.ds(0, K)], bufs[p],
                                  sems[p]).wait()
            pltpu.make_async_copy(wexp_hbm.at[0, 0], wrows[p],
                                  wsems[p]).wait()

        def compute(b, p):
            buf = bufs[p]
            wrow = wrows[p]

            def ebody(e, carry):
                sl = pl.ds(e * 16, 16)
                acc = bias_v[sl]
                for kk in range(K):
                    acc = acc + wrow[pl.ds(kk * LN_, LN_)] * buf[kk, sl]
                outbuf[p, sl] = acc
                return carry
            lax.fori_loop(0, EC, ebody, 0)
            pltpu.sync_copy(outbuf.at[p], recon_hbm.at[wid * bper + b])

        issue(0, 0)

        def body(t, carry):
            b0 = 2 * t
            issue(b0 + 1, 1)
            wait(0)
            compute(b0, 0)

            @pl.when(b0 + 2 < bper)
            def _():
                issue(b0 + 2, 0)
            wait(1)
            compute(b0 + 1, 1)
            return carry
        lax.fori_loop(0, bper // 2, body, 0)

    return k(W_dec, feats.reshape(NW, bper, K), feats.reshape(NW, K, bper),
             wexp.reshape(NW, bper, KL), bias)


def _sum_counts(counts2):
    F = counts2.shape[1]

    def body(c_ref, o_ref):
        o_ref[...] = c_ref[0:1, :] + c_ref[1:2, :]

    return pl.pallas_call(
        body,
        out_shape=jax.ShapeDtypeStruct((1, F), jnp.int32),
    )(counts2).reshape(F)


def kernel(embed, bias, W_enc, W_dec):
    B, E = embed.shape
    F = W_enc.shape[0]
    project, bids = _encode(embed, bias, W_enc)
    nbuk = F // G_
    fidx = (bids + jnp.arange(B, dtype=jnp.int32)[:, None] * nbuk).reshape(-1)
    NW, CH = 32, 128
    NCH = fidx.shape[0] // (NW * CH)
    cands_flat = _gather_cands(project.reshape(B * nbuk, G_),
                               fidx.reshape(NW, NCH, CH))
    cands = cands_flat.reshape(B, TOPK_ * G_)
    wexp, feats = _select(cands, bids)
    recon, counts2 = _decode(W_dec, feats, wexp, bias)
    total = _sum_counts(counts2)
    return recon, total
